# baseline (speedup 1.0000x reference)
import functools

import jax
import jax.numpy as jnp
from jax import lax
from jax.experimental import pallas as pl
from jax.experimental.pallas import tpu as pltpu
from jax.experimental.pallas import tpu_sc as plsc

N = 10000
E = 320000
D = 128
H = 256
O = 128

NCHUNK = 5
EC = E // NCHUNK
EDGE_BLOCK = 4000
NBLK = E // EDGE_BLOCK
NBLK_C = EC // EDGE_BLOCK
NBANKS = 8
GATHER_WINDOW = 256
IDXBLK_C = EC // GATHER_WINDOW
NEG = -3.0e38


def _sc_gather(h, edge_index, c):
  mesh = plsc.VectorSubcoreMesh(core_axis_name="core", subcore_axis_name="subcore")

  def idx_map(i):
    is_dst = i < IDXBLK_C
    row = jnp.where(is_dst, 1, 0)
    col = c * IDXBLK_C + jnp.where(is_dst, i, i - IDXBLK_C)
    return (row, col)

  @functools.partial(
      pl.kernel,
      out_type=jax.ShapeDtypeStruct((2 * EC, D), h.dtype),
      mesh=mesh,
  )
  def gather_kernel(h_hbm, i_hbm, o_hbm):
    def body(i_vmem, o_vmem):
      pltpu.sync_copy(h_hbm.at[i_vmem.at[0]], o_vmem)

    pltpu.emit_pipeline(
        body,
        grid=(2 * IDXBLK_C,),
        in_specs=[pl.BlockSpec((1, GATHER_WINDOW), index_map=idx_map)],
        out_specs=[pl.BlockSpec((GATHER_WINDOW, D), index_map=lambda i: (i, 0))],
        core_axis_name=("core", "subcore"),
        dimension_semantics=(pltpu.PARALLEL,),
    )(i_hbm, o_hbm)

  return gather_kernel(h, edge_index)


def _elu(x):
  return jnp.where(x > 0, x, jnp.exp(jnp.minimum(x, 0.0)) - 1.0)


def _edge_kernel(gd_ref, gs_ref, w1at_ref, w1bt_ref, w2t_ref, b1_ref, b2_ref,
                 dst_ref, out_ref, *scratch):
  banks = scratch[:NBANKS]
  m2_ref = scratch[NBANKS]
  i = pl.program_id(0)

  @pl.when(i == 0)
  def _init():
    for b in banks:
      b[...] = jnp.full(b.shape, NEG, jnp.float32)

  pre1 = (
      jnp.dot(gd_ref[...].astype(jnp.bfloat16), w1at_ref[...].astype(jnp.bfloat16),
              preferred_element_type=jnp.float32)
      + jnp.dot(gs_ref[...].astype(jnp.bfloat16), w1bt_ref[...].astype(jnp.bfloat16),
                preferred_element_type=jnp.float32)
      + b1_ref[...]
  )
  m1 = _elu(pre1).astype(jnp.bfloat16)
  z = (jnp.dot(m1, w2t_ref[...].astype(jnp.bfloat16),
               preferred_element_type=jnp.float32) + b2_ref[...])
  m2_ref[...] = z

  UNROLL = 50 * NBANKS

  def body(j, carry):
    base = pl.multiple_of(j * UNROLL, UNROLL)
    chunk = m2_ref[pl.ds(base, UNROLL), :]
    for k in range(UNROLL):
      idx = dst_ref[0, 0, j * UNROLL + k]
      row = chunk[k:k + 1, :]
      bank = banks[k % NBANKS]
      cur = bank[pl.ds(idx, 1), :]
      bank[pl.ds(idx, 1), :] = jnp.maximum(cur, row)
    return carry

  lax.fori_loop(0, EDGE_BLOCK // UNROLL, body, 0)

  @pl.when(i == NBLK_C - 1)
  def _finalize():
    m = banks[0][...]
    for k in range(1, NBANKS):
      m = jnp.maximum(m, banks[k][...])
    out_ref[...] = m


def _edge_partial(g, w1at, w1bt, w2t, b1r, b2r, eib, c):
  return pl.pallas_call(
      _edge_kernel,
      grid=(NBLK_C,),
      in_specs=[
          pl.BlockSpec((EDGE_BLOCK, D), lambda i: (i, 0)),
          pl.BlockSpec((EDGE_BLOCK, D), lambda i: (i + NBLK_C, 0)),
          pl.BlockSpec((D, H), lambda i: (0, 0)),
          pl.BlockSpec((D, H), lambda i: (0, 0)),
          pl.BlockSpec((H, O), lambda i: (0, 0)),
          pl.BlockSpec((1, H), lambda i: (0, 0)),
          pl.BlockSpec((1, O), lambda i: (0, 0)),
          pl.BlockSpec((1, 1, EDGE_BLOCK), lambda i: (NBLK + c * NBLK_C + i, 0, 0),
                       memory_space=pltpu.MemorySpace.SMEM),
      ],
      out_specs=pl.BlockSpec((N, O), lambda i: (0, 0)),
      out_shape=jax.ShapeDtypeStruct((N, O), jnp.float32),
      scratch_shapes=(
          [pltpu.VMEM((N, O), jnp.float32) for _ in range(NBANKS)]
          + [pltpu.VMEM((EDGE_BLOCK, O), jnp.float32)]
      ),
      compiler_params=pltpu.CompilerParams(
          dimension_semantics=("arbitrary",),
          vmem_limit_bytes=100 * 1024 * 1024,
      ),
  )(g, g, w1at, w1bt, w2t, b1r, b2r, eib)


def _combine_kernel(*refs):
  parts = refs[:NCHUNK]
  out_ref = refs[NCHUNK]
  m = parts[0][...]
  for c in range(1, NCHUNK):
    m = jnp.maximum(m, parts[c][...])
  out_ref[...] = jnp.where(m < -1.0e38, 0.0, _elu(m))


def kernel(h, edge_index, W1, b1, W2, b2):
  w1at = W1[:, :D].T
  w1bt = W1[:, D:].T
  w2t = W2.T
  b1r = b1.reshape(1, H)
  b2r = b2.reshape(1, O)
  eib = edge_index.reshape(2 * NBLK, 1, EDGE_BLOCK)

  partials = []
  for c in range(NCHUNK):
    g = _sc_gather(h, edge_index, c)
    partials.append(_edge_partial(g, w1at, w1bt, w2t, b1r, b2r, eib, c))

  out = pl.pallas_call(
      _combine_kernel,
      grid=(1,),
      in_specs=[pl.BlockSpec((N, O), lambda i: (0, 0)) for _ in range(NCHUNK)],
      out_specs=pl.BlockSpec((N, O), lambda i: (0, 0)),
      out_shape=jax.ShapeDtypeStruct((N, O), jnp.float32),
  )(*partials)
  return out


# speedup vs baseline: 1.0224x; 1.0224x over previous
import functools

import jax
import jax.numpy as jnp
from jax import lax
from jax.experimental import pallas as pl
from jax.experimental.pallas import tpu as pltpu
from jax.experimental.pallas import tpu_sc as plsc

N = 10000
E = 320000
D = 128
H = 256
O = 128

CHUNKS = (32000, 64000, 64000, 64000, 96000)
EDGE_BLOCK = 4000
NBLK = E // EDGE_BLOCK
NBANKS = 8
UNROLL = 400
GATHER_WINDOW = 256
NEG = -3.0e38


def _sc_gather(h, edge_index, off, ec):
  mesh = plsc.VectorSubcoreMesh(core_axis_name="core", subcore_axis_name="subcore")
  nwin = ec // GATHER_WINDOW
  base = off // GATHER_WINDOW

  def idx_map(i):
    is_dst = i < nwin
    row = jnp.where(is_dst, 1, 0)
    col = base + jnp.where(is_dst, i, i - nwin)
    return (row, col)

  @functools.partial(
      pl.kernel,
      out_type=jax.ShapeDtypeStruct((2 * ec, D), h.dtype),
      mesh=mesh,
  )
  def gather_kernel(h_hbm, i_hbm, o_hbm):
    def body(i_vmem, o_vmem):
      pltpu.sync_copy(h_hbm.at[i_vmem.at[0]], o_vmem)

    pltpu.emit_pipeline(
        body,
        grid=(2 * nwin,),
        in_specs=[pl.BlockSpec((1, GATHER_WINDOW), index_map=idx_map)],
        out_specs=[pl.BlockSpec((GATHER_WINDOW, D), index_map=lambda i: (i, 0))],
        core_axis_name=("core", "subcore"),
        dimension_semantics=(pltpu.PARALLEL,),
    )(i_hbm, o_hbm)

  return gather_kernel(h, edge_index)


def _elu(x):
  return jnp.where(x > 0, x, jnp.exp(jnp.minimum(x, 0.0)) - 1.0)


def _make_edge_kernel(nblk_c, is_last):
  def _edge_kernel(gd_ref, gs_ref, w1at_ref, w1bt_ref, w2t_ref, b1_ref, b2_ref,
                   dst_ref, prev_ref, out_ref, *scratch):
    banks = scratch[:NBANKS]
    m2_ref = scratch[NBANKS]
    i = pl.program_id(0)

    @pl.when(i == 0)
    def _init():
      for b in banks:
        b[...] = jnp.full(b.shape, NEG, jnp.float32)

    pre1 = (
        jnp.dot(gd_ref[...], w1at_ref[...], preferred_element_type=jnp.float32)
        + jnp.dot(gs_ref[...], w1bt_ref[...], preferred_element_type=jnp.float32)
        + b1_ref[...]
    )
    m1 = _elu(pre1)
    z = jnp.dot(m1, w2t_ref[...], preferred_element_type=jnp.float32) + b2_ref[...]
    m2_ref[...] = z

    def body(j, carry):
      base = pl.multiple_of(j * UNROLL, UNROLL)
      chunk = m2_ref[pl.ds(base, UNROLL), :]
      for k in range(UNROLL):
        idx = dst_ref[0, 0, j * UNROLL + k]
        row = chunk[k:k + 1, :]
        bank = banks[k % NBANKS]
        cur = bank[pl.ds(idx, 1), :]
        bank[pl.ds(idx, 1), :] = jnp.maximum(cur, row)
      return carry

    lax.fori_loop(0, EDGE_BLOCK // UNROLL, body, 0)

    @pl.when(i == nblk_c - 1)
    def _finalize():
      m = prev_ref[...]
      for k in range(NBANKS):
        m = jnp.maximum(m, banks[k][...])
      if is_last:
        m = jnp.where(m < -1.0e38, 0.0, _elu(m))
      out_ref[...] = m

  return _edge_kernel


def _edge_partial(g, w1at, w1bt, w2t, b1r, b2r, eib, prev, off, ec, is_last):
  nblk_c = ec // EDGE_BLOCK
  boff = off // EDGE_BLOCK
  return pl.pallas_call(
      _make_edge_kernel(nblk_c, is_last),
      grid=(nblk_c,),
      in_specs=[
          pl.BlockSpec((EDGE_BLOCK, D), lambda i: (i, 0)),
          pl.BlockSpec((EDGE_BLOCK, D), lambda i: (i + nblk_c, 0)),
          pl.BlockSpec((D, H), lambda i: (0, 0)),
          pl.BlockSpec((D, H), lambda i: (0, 0)),
          pl.BlockSpec((H, O), lambda i: (0, 0)),
          pl.BlockSpec((1, H), lambda i: (0, 0)),
          pl.BlockSpec((1, O), lambda i: (0, 0)),
          pl.BlockSpec((1, 1, EDGE_BLOCK), lambda i: (NBLK + boff + i, 0, 0),
                       memory_space=pltpu.MemorySpace.SMEM),
          pl.BlockSpec((N, O), lambda i: (0, 0)),
      ],
      out_specs=pl.BlockSpec((N, O), lambda i: (0, 0)),
      out_shape=jax.ShapeDtypeStruct((N, O), jnp.float32),
      scratch_shapes=(
          [pltpu.VMEM((N, O), jnp.float32) for _ in range(NBANKS)]
          + [pltpu.VMEM((EDGE_BLOCK, O), jnp.float32)]
      ),
      compiler_params=pltpu.CompilerParams(
          dimension_semantics=("arbitrary",),
          vmem_limit_bytes=100 * 1024 * 1024,
      ),
  )(g, g, w1at, w1bt, w2t, b1r, b2r, eib, prev)


def kernel(h, edge_index, W1, b1, W2, b2):
  w1at = W1[:, :D].T
  w1bt = W1[:, D:].T
  w2t = W2.T
  b1r = b1.reshape(1, H)
  b2r = b2.reshape(1, O)
  eib = edge_index.reshape(2 * NBLK, 1, EDGE_BLOCK)

  gathers = []
  off = 0
  for ec in CHUNKS:
    gathers.append((_sc_gather(h, edge_index, off, ec), off, ec))
    off += ec

  running = jnp.full((N, O), NEG, jnp.float32)
  for c, (g, off, ec) in enumerate(gathers):
    running = _edge_partial(g, w1at, w1bt, w2t, b1r, b2r, eib, running,
                            off, ec, c == len(CHUNKS) - 1)
  return running
